# baseline (speedup 1.0000x reference)
import jax
import jax.numpy as jnp
from jax.experimental import pallas as pl
from jax.experimental.pallas import tpu as pltpu

_N = 4096
_D = 512
_BM = 512
_KC = 4
_BK = _N // _KC


def _gcn_body(h_ref, w_ref, adj_ref, b_ref, out_ref, sup_ref):
    i = pl.program_id(0)

    @pl.when(i == 0)
    def _support():
        hb = h_ref[...].astype(jnp.bfloat16)
        wb = w_ref[...].astype(jnp.bfloat16)
        sup = jnp.dot(hb, wb, preferred_element_type=jnp.float32)
        sup_ref[...] = sup.astype(jnp.bfloat16)

    @pl.when(i > 0)
    def _rows():
        acc = b_ref[...].astype(jnp.float32)
        for c in range(_KC):
            ab = adj_ref[:, c * _BK:(c + 1) * _BK].astype(jnp.bfloat16)
            acc = acc + jnp.dot(ab, sup_ref[c * _BK:(c + 1) * _BK, :],
                                preferred_element_type=jnp.float32)
        out_ref[...] = jnp.maximum(acc, 0.0)


def kernel(h, adj, W, b):
    b2 = b.reshape(1, _D)
    row = lambda i: (jnp.maximum(i - 1, 0), 0)
    return pl.pallas_call(
        _gcn_body,
        grid=(_N // _BM + 1,),
        in_specs=[
            pl.BlockSpec((_N, _D), lambda i: (0, 0)),
            pl.BlockSpec((_D, _D), lambda i: (0, 0)),
            pl.BlockSpec((_BM, _N), row),
            pl.BlockSpec((1, _D), lambda i: (0, 0)),
        ],
        out_specs=pl.BlockSpec((_BM, _D), row),
        out_shape=jax.ShapeDtypeStruct((_N, _D), jnp.float32),
        scratch_shapes=[pltpu.VMEM((_N, _D), jnp.bfloat16)],
        compiler_params=pltpu.CompilerParams(
            dimension_semantics=("arbitrary",),
        ),
    )(h, W, adj, b2)


# speedup vs baseline: 1.0008x; 1.0008x over previous
import jax
import jax.numpy as jnp
from jax.experimental import pallas as pl
from jax.experimental.pallas import tpu as pltpu

_N = 4096
_D = 512
_BM = 256
_H = _N // 2


def _gcn_body(h_ref, w_ref, adj_ref, b_ref, out_ref, sup_ref):
    i = pl.program_id(0)

    @pl.when(i == 0)
    def _support():
        hb = h_ref[...].astype(jnp.bfloat16)
        wb = w_ref[...].astype(jnp.bfloat16)
        sup = jnp.dot(hb, wb, preferred_element_type=jnp.float32)
        sup_ref[...] = sup.astype(jnp.bfloat16)

    @pl.when(i > 0)
    def _rows():
        bias = b_ref[...]
        for half in range(2):
            ab = adj_ref[half].astype(jnp.bfloat16)
            acc = jnp.dot(ab, sup_ref[...], preferred_element_type=jnp.float32)
            out_ref[half] = jnp.maximum(acc + bias, 0.0)


def kernel(h, adj, W, b):
    b2 = b.reshape(1, _D)
    adj3 = adj.reshape(2, _H, _N)
    row = lambda i: (0, jnp.maximum(i - 1, 0), 0)
    out = pl.pallas_call(
        _gcn_body,
        grid=(_H // _BM + 1,),
        in_specs=[
            pl.BlockSpec((_N, _D), lambda i: (0, 0)),
            pl.BlockSpec((_D, _D), lambda i: (0, 0)),
            pl.BlockSpec((2, _BM, _N), row),
            pl.BlockSpec((1, _D), lambda i: (0, 0)),
        ],
        out_specs=pl.BlockSpec((2, _BM, _D), row),
        out_shape=jax.ShapeDtypeStruct((2, _H, _D), jnp.float32),
        scratch_shapes=[pltpu.VMEM((_N, _D), jnp.bfloat16)],
        compiler_params=pltpu.CompilerParams(
            dimension_semantics=("arbitrary",),
        ),
    )(h, W, adj3, b2)
    return out.reshape(_N, _D)
